# baseline (speedup 1.0000x reference)
import functools

import numpy as np

import jax
import jax.numpy as jnp
from jax import lax
from jax.experimental import pallas as pl
from jax.experimental.pallas import tpu as pltpu
from jax.experimental.pallas import tpu_sc as plsc

_B = 128
_V = 100000
_K = 1024
_NW = 32
_R = _B // _NW
_L = 16

_ROWS_PER_BLOCK = 8
_NUM_BLOCKS = _B // _ROWS_PER_BLOCK


@functools.cache
def _consts():
    with jax.ensure_compile_time_eval():
        return _consts_impl()


def _consts_impl():
    key = jax.random.key(42)
    u = jax.random.uniform(key, (_B, _V), dtype=jnp.float32,
                           minval=1e-20, maxval=1.0)
    gumbel = -jnp.log(-jnp.log(u))
    g = np.asarray(gumbel)
    topv, topi = jax.lax.top_k(gumbel, _K + 1)
    topv, topi = np.asarray(topv), np.asarray(topi)
    cand = np.sort(topi[:, :_K], axis=1).astype(np.int32)
    g_sub = topv[:, _K]
    cand_g = np.take_along_axis(g, cand, axis=1).astype(np.float32)
    return dict(cand_cols=jnp.asarray(cand),
                cand_g=jnp.asarray(cand_g),
                gsub=jnp.asarray(g_sub.astype(np.float32)))


def _rowmax_body(logits_ref, out_ref):
    out_ref[0, 0, :] = jnp.max(logits_ref[...], axis=-1)


def _rowmax(logits):
    out = pl.pallas_call(
        _rowmax_body,
        grid=(_NUM_BLOCKS,),
        in_specs=[pl.BlockSpec((_ROWS_PER_BLOCK, _V), lambda i: (i, 0))],
        out_specs=pl.BlockSpec((1, 1, _ROWS_PER_BLOCK), lambda i: (i, 0, 0)),
        out_shape=jax.ShapeDtypeStruct((_NUM_BLOCKS, 1, _ROWS_PER_BLOCK),
                                       jnp.float32),
    )(logits)
    return out.reshape(_B)


@functools.cache
def _make_sc():
    mesh = plsc.VectorSubcoreMesh(core_axis_name="c", subcore_axis_name="s")

    @functools.partial(
        pl.kernel,
        out_type=(jax.ShapeDtypeStruct((_NW, _R, _L), jnp.float32),
                  jax.ShapeDtypeStruct((_NW, _R, _L), jnp.int32)),
        mesh=mesh,
        compiler_params=pltpu.CompilerParams(use_tc_tiling_on_sc=False,
                                             needs_layout_passes=False),
        scratch_types=[
            pltpu.VMEM((_R, _K), jnp.float32),
            pltpu.VMEM((_R, _K), jnp.float32),
            pltpu.VMEM((_R, _K), jnp.int32),
            pltpu.VMEM((_R, _L), jnp.float32),
            pltpu.VMEM((_R, _L), jnp.int32),
        ],
    )
    def sc_fn(cv, cg, cj, out_mc, out_j, cv_v, g_v, j_v, rmc_v, rj_v):
        wid = lax.axis_index("s") * 2 + lax.axis_index("c")
        base = wid * _R
        pltpu.sync_copy(cv.at[pl.ds(base, _R)], cv_v)
        pltpu.sync_copy(cg.at[pl.ds(base, _R)], g_v)
        pltpu.sync_copy(cj.at[pl.ds(base, _R)], j_v)
        for r4 in range(_R):
            m = jnp.full((_L,), -jnp.inf, jnp.float32)
            jx = jnp.full((_L,), _V, jnp.int32)
            for p0 in range(0, _K, _L):
                s = cv_v[r4, pl.ds(p0, _L)] + g_v[r4, pl.ds(p0, _L)]
                cjv = j_v[r4, pl.ds(p0, _L)]
                upd = s > m
                m = jnp.where(upd, s, m)
                jx = jnp.where(upd, cjv, jx)
            mc_row = jnp.max(m)
            j_row = jnp.min(jnp.where(m == mc_row, jx, _V))
            rmc_v[r4, :] = jnp.full((_L,), mc_row, jnp.float32)
            rj_v[r4, :] = jnp.full((_L,), j_row, jnp.int32)
        pltpu.sync_copy(rmc_v, out_mc.at[wid])
        pltpu.sync_copy(rj_v, out_j.at[wid])

    return sc_fn


def _argmax_body(logits_ref, gumbel_ref, out_ref):
    x = logits_ref[...] + gumbel_ref[...]
    m = jnp.max(x, axis=-1, keepdims=True)
    iota = jax.lax.broadcasted_iota(jnp.int32, x.shape, 1)
    idx = jnp.min(jnp.where(x == m, iota, _V), axis=-1)
    out_ref[0, 0, :] = idx


def _dense_fallback(logits):
    key = jax.random.key(42)
    u = jax.random.uniform(key, logits.shape, dtype=logits.dtype,
                           minval=1e-20, maxval=1.0)
    gumbel = -jnp.log(-jnp.log(u))
    return _dense_argmax(logits, gumbel)


def _dense_argmax(logits, gumbel):
    out = pl.pallas_call(
        _argmax_body,
        grid=(_NUM_BLOCKS,),
        in_specs=[
            pl.BlockSpec((_ROWS_PER_BLOCK, _V), lambda i: (i, 0)),
            pl.BlockSpec((_ROWS_PER_BLOCK, _V), lambda i: (i, 0)),
        ],
        out_specs=pl.BlockSpec((1, 1, _ROWS_PER_BLOCK), lambda i: (i, 0, 0)),
        out_shape=jax.ShapeDtypeStruct((_NUM_BLOCKS, 1, _ROWS_PER_BLOCK),
                                       jnp.int32),
    )(logits, gumbel)
    return out.reshape(_B)


def kernel(logits):
    cs = _consts()
    m_row = _rowmax(logits)
    cv = jnp.take_along_axis(logits, cs["cand_cols"], axis=1)
    out_mc, out_j = _make_sc()(cv, cs["cand_g"], cs["cand_cols"])
    mc = out_mc[:, :, 0].reshape(_B)
    j_fast = out_j[:, :, 0].reshape(_B)
    ok = jnp.all(mc > m_row + cs["gsub"])
    idx = lax.cond(ok, lambda: j_fast, lambda: _dense_fallback(logits))
    return idx.astype(jnp.int64)


# speedup vs baseline: 1.6718x; 1.6718x over previous
import functools

import numpy as np

import jax
import jax.numpy as jnp
from jax import lax
from jax.experimental import pallas as pl
from jax.experimental.pallas import tpu as pltpu
from jax.experimental.pallas import tpu_sc as plsc

_B = 128
_V = 100000
_K = 1024
_NW = 32
_R = _B // _NW
_L = 16

_ROWS_PER_BLOCK = 8
_NUM_BLOCKS = _B // _ROWS_PER_BLOCK


@functools.cache
def _consts():
    with jax.ensure_compile_time_eval():
        return _consts_impl()


def _consts_impl():
    key = jax.random.key(42)
    u = jax.random.uniform(key, (_B, _V), dtype=jnp.float32,
                           minval=1e-20, maxval=1.0)
    gumbel = -jnp.log(-jnp.log(u))
    g = np.asarray(gumbel)
    topv, topi = jax.lax.top_k(gumbel, _K + 1)
    topv, topi = np.asarray(topv), np.asarray(topi)
    cand = np.sort(topi[:, :_K], axis=1).astype(np.int32)
    g_sub = topv[:, _K]
    cand_g = np.take_along_axis(g, cand, axis=1).astype(np.float32)
    return dict(cand_cols=jnp.asarray(cand),
                cand_g=jnp.asarray(cand_g),
                gsub=jnp.asarray(g_sub.astype(np.float32)))


_CB = 4000


def _rowmax_t_body(x_ref, out_ref):
    i = pl.program_id(0)
    m = jnp.max(x_ref[...], axis=0, keepdims=True)

    @pl.when(i == 0)
    def _init():
        out_ref[...] = m

    @pl.when(i > 0)
    def _acc():
        out_ref[...] = jnp.maximum(out_ref[...], m)


def _rowmax_t(lt):
    out = pl.pallas_call(
        _rowmax_t_body,
        grid=(_V // _CB,),
        in_specs=[pl.BlockSpec((_CB, _B), lambda i: (i, 0))],
        out_specs=pl.BlockSpec((1, _B), lambda i: (0, 0)),
        out_shape=jax.ShapeDtypeStruct((1, _B), jnp.float32),
    )(lt)
    return out.reshape(_B)


@functools.cache
def _make_sc():
    mesh = plsc.VectorSubcoreMesh(core_axis_name="c", subcore_axis_name="s")

    @functools.partial(
        pl.kernel,
        out_type=(jax.ShapeDtypeStruct((_NW, _R, _L), jnp.float32),
                  jax.ShapeDtypeStruct((_NW, _R, _L), jnp.int32)),
        mesh=mesh,
        compiler_params=pltpu.CompilerParams(use_tc_tiling_on_sc=False,
                                             needs_layout_passes=False),
        scratch_types=[
            pltpu.VMEM((_R, _K), jnp.float32),
            pltpu.VMEM((_R, _K), jnp.float32),
            pltpu.VMEM((_R, _K), jnp.int32),
            pltpu.VMEM((_R, _L), jnp.float32),
            pltpu.VMEM((_R, _L), jnp.int32),
        ],
    )
    def sc_fn(cv, cg, cj, out_mc, out_j, cv_v, g_v, j_v, rmc_v, rj_v):
        wid = lax.axis_index("s") * 2 + lax.axis_index("c")
        base = wid * _R
        pltpu.sync_copy(cv.at[pl.ds(base, _R)], cv_v)
        pltpu.sync_copy(cg.at[pl.ds(base, _R)], g_v)
        pltpu.sync_copy(cj.at[pl.ds(base, _R)], j_v)
        for r4 in range(_R):
            m = jnp.full((_L,), -jnp.inf, jnp.float32)
            jx = jnp.full((_L,), _V, jnp.int32)
            for p0 in range(0, _K, _L):
                s = cv_v[r4, pl.ds(p0, _L)] + g_v[r4, pl.ds(p0, _L)]
                cjv = j_v[r4, pl.ds(p0, _L)]
                upd = s > m
                m = jnp.where(upd, s, m)
                jx = jnp.where(upd, cjv, jx)
            mc_row = jnp.max(m)
            j_row = jnp.min(jnp.where(m == mc_row, jx, _V))
            rmc_v[r4, :] = jnp.full((_L,), mc_row, jnp.float32)
            rj_v[r4, :] = jnp.full((_L,), j_row, jnp.int32)
        pltpu.sync_copy(rmc_v, out_mc.at[wid])
        pltpu.sync_copy(rj_v, out_j.at[wid])

    return sc_fn


def _fb_body(x_ref, g_ref, idx_ref, m_ref):
    i = pl.program_id(0)
    s = x_ref[...] + g_ref[...]
    m = jnp.max(s, axis=0, keepdims=True)
    iota = jax.lax.broadcasted_iota(jnp.int32, s.shape, 0) + i * _CB
    idx = jnp.min(jnp.where(s == m, iota, _V), axis=0, keepdims=True)

    @pl.when(i == 0)
    def _init():
        m_ref[...] = m
        idx_ref[...] = idx

    @pl.when(i > 0)
    def _acc():
        better = m > m_ref[...]
        equal = m == m_ref[...]
        idx_ref[...] = jnp.where(
            better, idx,
            jnp.where(equal, jnp.minimum(idx, idx_ref[...]), idx_ref[...]))
        m_ref[...] = jnp.maximum(m_ref[...], m)


def _dense_fallback(lt):
    key = jax.random.key(42)
    u = jax.random.uniform(key, (_B, _V), dtype=jnp.float32,
                           minval=1e-20, maxval=1.0)
    gt = (-jnp.log(-jnp.log(u))).T
    idx, _ = pl.pallas_call(
        _fb_body,
        grid=(_V // _CB,),
        in_specs=[
            pl.BlockSpec((_CB, _B), lambda i: (i, 0)),
            pl.BlockSpec((_CB, _B), lambda i: (i, 0)),
        ],
        out_specs=(pl.BlockSpec((1, _B), lambda i: (0, 0)),
                   pl.BlockSpec((1, _B), lambda i: (0, 0))),
        out_shape=(jax.ShapeDtypeStruct((1, _B), jnp.int32),
                   jax.ShapeDtypeStruct((1, _B), jnp.float32)),
    )(lt, gt)
    return idx.reshape(_B)


def kernel(logits):
    cs = _consts()
    lt = logits.T
    m_row = _rowmax_t(lt)
    cv = jnp.take_along_axis(logits, cs["cand_cols"], axis=1)
    out_mc, out_j = _make_sc()(cv, cs["cand_g"], cs["cand_cols"])
    mc = out_mc[:, :, 0].reshape(_B)
    j_fast = out_j[:, :, 0].reshape(_B)
    ok = jnp.all(mc > m_row + cs["gsub"])
    idx = lax.cond(ok, lambda: j_fast, lambda: _dense_fallback(lt))
    return idx.astype(jnp.int64)


# speedup vs baseline: 1.8276x; 1.0932x over previous
import functools

import numpy as np

import jax
import jax.numpy as jnp
from jax import lax
from jax.experimental import pallas as pl
from jax.experimental.pallas import tpu as pltpu
from jax.experimental.pallas import tpu_sc as plsc

_B = 128
_V = 100000
_K = 512
_NW = 32
_R = _B // _NW
_L = 16

_ROWS_PER_BLOCK = 8
_NUM_BLOCKS = _B // _ROWS_PER_BLOCK


@functools.cache
def _consts():
    with jax.ensure_compile_time_eval():
        return _consts_impl()


def _consts_impl():
    key = jax.random.key(42)
    u = jax.random.uniform(key, (_B, _V), dtype=jnp.float32,
                           minval=1e-20, maxval=1.0)
    gumbel = -jnp.log(-jnp.log(u))
    g = np.asarray(gumbel)
    topv, topi = jax.lax.top_k(gumbel, _K + 1)
    topv, topi = np.asarray(topv), np.asarray(topi)
    cand = np.sort(topi[:, :_K], axis=1).astype(np.int32)
    g_sub = topv[:, _K]
    cand_g = np.take_along_axis(g, cand, axis=1).astype(np.float32)
    return dict(cand_cols=jnp.asarray(cand),
                cand_g=jnp.asarray(cand_g),
                gsub=jnp.asarray(g_sub.astype(np.float32)))


_CB = 5000


def _rowmax_t_body(x_ref, out_ref):
    i = pl.program_id(0)
    m = jnp.max(x_ref[...], axis=0, keepdims=True)

    @pl.when(i == 0)
    def _init():
        out_ref[...] = m

    @pl.when(i > 0)
    def _acc():
        out_ref[...] = jnp.maximum(out_ref[...], m)


def _rowmax_t(lt):
    out = pl.pallas_call(
        _rowmax_t_body,
        grid=(_V // _CB,),
        in_specs=[pl.BlockSpec((_CB, _B), lambda i: (i, 0))],
        out_specs=pl.BlockSpec((1, _B), lambda i: (0, 0)),
        out_shape=jax.ShapeDtypeStruct((1, _B), jnp.float32),
    )(lt)
    return out.reshape(_B)


@functools.cache
def _make_sc():
    mesh = plsc.VectorSubcoreMesh(core_axis_name="c", subcore_axis_name="s")

    @functools.partial(
        pl.kernel,
        out_type=(jax.ShapeDtypeStruct((_NW, _R, _L), jnp.float32),
                  jax.ShapeDtypeStruct((_NW, _R, _L), jnp.int32)),
        mesh=mesh,
        compiler_params=pltpu.CompilerParams(use_tc_tiling_on_sc=False,
                                             needs_layout_passes=False),
        scratch_types=[
            pltpu.VMEM((_R, _K), jnp.float32),
            pltpu.VMEM((_R, _K), jnp.float32),
            pltpu.VMEM((_R, _K), jnp.int32),
            pltpu.VMEM((_R, _L), jnp.float32),
            pltpu.VMEM((_R, _L), jnp.int32),
        ],
    )
    def sc_fn(cv, cg, cj, out_mc, out_j, cv_v, g_v, j_v, rmc_v, rj_v):
        wid = lax.axis_index("s") * 2 + lax.axis_index("c")
        base = wid * _R
        pltpu.sync_copy(cv.at[pl.ds(base, _R)], cv_v)
        pltpu.sync_copy(cg.at[pl.ds(base, _R)], g_v)
        pltpu.sync_copy(cj.at[pl.ds(base, _R)], j_v)
        for r4 in range(_R):
            m = jnp.full((_L,), -jnp.inf, jnp.float32)
            jx = jnp.full((_L,), _V, jnp.int32)
            for p0 in range(0, _K, _L):
                s = cv_v[r4, pl.ds(p0, _L)] + g_v[r4, pl.ds(p0, _L)]
                cjv = j_v[r4, pl.ds(p0, _L)]
                upd = s > m
                m = jnp.where(upd, s, m)
                jx = jnp.where(upd, cjv, jx)
            mc_row = jnp.max(m)
            j_row = jnp.min(jnp.where(m == mc_row, jx, _V))
            rmc_v[r4, :] = jnp.full((_L,), mc_row, jnp.float32)
            rj_v[r4, :] = jnp.full((_L,), j_row, jnp.int32)
        pltpu.sync_copy(rmc_v, out_mc.at[wid])
        pltpu.sync_copy(rj_v, out_j.at[wid])

    return sc_fn


def _fb_body(x_ref, g_ref, idx_ref, m_ref):
    i = pl.program_id(0)
    s = x_ref[...] + g_ref[...]
    m = jnp.max(s, axis=0, keepdims=True)
    iota = jax.lax.broadcasted_iota(jnp.int32, s.shape, 0) + i * _CB
    idx = jnp.min(jnp.where(s == m, iota, _V), axis=0, keepdims=True)

    @pl.when(i == 0)
    def _init():
        m_ref[...] = m
        idx_ref[...] = idx

    @pl.when(i > 0)
    def _acc():
        better = m > m_ref[...]
        equal = m == m_ref[...]
        idx_ref[...] = jnp.where(
            better, idx,
            jnp.where(equal, jnp.minimum(idx, idx_ref[...]), idx_ref[...]))
        m_ref[...] = jnp.maximum(m_ref[...], m)


def _dense_fallback(lt):
    key = jax.random.key(42)
    u = jax.random.uniform(key, (_B, _V), dtype=jnp.float32,
                           minval=1e-20, maxval=1.0)
    gt = (-jnp.log(-jnp.log(u))).T
    idx, _ = pl.pallas_call(
        _fb_body,
        grid=(_V // _CB,),
        in_specs=[
            pl.BlockSpec((_CB, _B), lambda i: (i, 0)),
            pl.BlockSpec((_CB, _B), lambda i: (i, 0)),
        ],
        out_specs=(pl.BlockSpec((1, _B), lambda i: (0, 0)),
                   pl.BlockSpec((1, _B), lambda i: (0, 0))),
        out_shape=(jax.ShapeDtypeStruct((1, _B), jnp.int32),
                   jax.ShapeDtypeStruct((1, _B), jnp.float32)),
    )(lt, gt)
    return idx.reshape(_B)


def kernel(logits):
    cs = _consts()
    lt = logits.T
    m_row = _rowmax_t(lt)
    cv = jnp.take_along_axis(logits, cs["cand_cols"], axis=1)
    out_mc, out_j = _make_sc()(cv, cs["cand_g"], cs["cand_cols"])
    mc = out_mc[:, :, 0].reshape(_B)
    j_fast = out_j[:, :, 0].reshape(_B)
    ok = jnp.all(mc > m_row + cs["gsub"])
    idx = lax.cond(ok, lambda: j_fast, lambda: _dense_fallback(lt))
    return idx.astype(jnp.int64)
